# baseline (speedup 1.0000x reference)
import functools

import jax
import jax.numpy as jnp
from jax import lax
from jax.experimental import pallas as pl
from jax.experimental.pallas import tpu as pltpu
from jax.experimental.pallas import tpu_sc as plsc

NUM_EMBEDDINGS = 100000
EMBEDDING_DIM = 128
BATCH = 4096
N_FIELDS = 26

_NC = 2
_NS = 16
_NW = _NC * _NS

_ROWS_PER_CHUNK = 4
_CHUNK = _ROWS_PER_CHUNK * N_FIELDS
_BATCH_PER_W = BATCH // _NW
_IDX_PER_W = _BATCH_PER_W * N_FIELDS
_CHUNKS_PER_W = _BATCH_PER_W // _ROWS_PER_CHUNK
_NBUF = 4
_MAIN = _CHUNKS_PER_W // _NBUF


@functools.partial(
    pl.kernel,
    out_type=jax.ShapeDtypeStruct((BATCH, N_FIELDS, EMBEDDING_DIM), jnp.float32),
    mesh=plsc.VectorSubcoreMesh(core_axis_name="c", subcore_axis_name="s"),
    scratch_types=[
        pltpu.VMEM((_IDX_PER_W,), jnp.int32),
        pltpu.VMEM((_NBUF, _CHUNK, EMBEDDING_DIM), jnp.float32),
        pltpu.SemaphoreType.DMA((_NBUF,)),
        pltpu.SemaphoreType.DMA((_NBUF,)),
    ],
)
def _gather_kernel(idx_hbm, table_hbm, out_hbm, idx_v, bufs, gsems, ssems):
    wid = lax.axis_index("s") * _NC + lax.axis_index("c")
    pltpu.sync_copy(idx_hbm.at[wid], idx_v)
    batch_base = wid * _BATCH_PER_W

    def g_start(j, b):
        pltpu.async_copy(
            table_hbm.at[idx_v.at[pl.ds(j * _CHUNK, _CHUNK)]],
            bufs.at[b], gsems.at[b])

    def g_wait(j, b):
        pltpu.make_async_copy(
            table_hbm.at[idx_v.at[pl.ds(j * _CHUNK, _CHUNK)]],
            bufs.at[b], gsems.at[b]).wait()

    def s_descr(j, b):
        src = bufs.at[b].reshape(_ROWS_PER_CHUNK, N_FIELDS, EMBEDDING_DIM)
        dst = out_hbm.at[pl.ds(batch_base + j * _ROWS_PER_CHUNK, _ROWS_PER_CHUNK)]
        return src, dst

    def s_start(j, b):
        src, dst = s_descr(j, b)
        pltpu.async_copy(src, dst, ssems.at[b])

    def s_wait(j, b):
        src, dst = s_descr(j, b)
        pltpu.make_async_copy(src, dst, ssems.at[b]).wait()

    for b in range(_NBUF):
        g_start(b, b)

    def outer(t, _):
        for b in range(_NBUF):
            j = t * _NBUF + b
            g_wait(j, b)
            s_start(j, b)
            jn = j + _NBUF

            @pl.when(jn < _CHUNKS_PER_W)
            def _():
                s_wait(j, b)
                g_start(jn, b)

        return 0

    lax.fori_loop(0, _MAIN, outer, 0)
    for j in range(_CHUNKS_PER_W - _NBUF, _CHUNKS_PER_W):
        s_wait(j, j % _NBUF)


def kernel(token_ids, weight):
    idx = token_ids.reshape(_NW, _IDX_PER_W).astype(jnp.int32)
    return _gather_kernel(idx, weight)


# speedup vs baseline: 1.8110x; 1.8110x over previous
import functools

import jax
import jax.numpy as jnp
from jax import lax
from jax.experimental import pallas as pl
from jax.experimental.pallas import tpu as pltpu
from jax.experimental.pallas import tpu_sc as plsc

NUM_EMBEDDINGS = 100000
EMBEDDING_DIM = 128
BATCH = 4096
N_FIELDS = 26

_NC = 2
_NS = 16
_NW = _NC * _NS

_CHUNK = 256
_TOTAL = BATCH * N_FIELDS
_IDX_PER_W = _TOTAL // _NW
_CHUNKS_PER_W = _IDX_PER_W // _CHUNK
_NBUF = 2


@functools.partial(
    pl.kernel,
    out_type=jax.ShapeDtypeStruct((_TOTAL, EMBEDDING_DIM), jnp.float32),
    mesh=plsc.VectorSubcoreMesh(core_axis_name="c", subcore_axis_name="s"),
    scratch_types=[
        pltpu.VMEM((_IDX_PER_W,), jnp.int32),
        pltpu.VMEM((_NBUF, _CHUNK, EMBEDDING_DIM), jnp.float32),
        pltpu.SemaphoreType.DMA((_NBUF,)),
        pltpu.SemaphoreType.DMA((_NBUF,)),
    ],
)
def _gather_kernel(idx_hbm, table_hbm, out_hbm, idx_v, bufs, gsems, ssems):
    wid = lax.axis_index("s") * _NC + lax.axis_index("c")
    pltpu.sync_copy(idx_hbm.at[wid], idx_v)
    row_base = wid * _IDX_PER_W

    def g_start(j, b):
        pltpu.async_copy(
            table_hbm.at[idx_v.at[pl.ds(j * _CHUNK, _CHUNK)]],
            bufs.at[b], gsems.at[b])

    def g_wait(j, b):
        pltpu.make_async_copy(
            table_hbm.at[idx_v.at[pl.ds(j * _CHUNK, _CHUNK)]],
            bufs.at[b], gsems.at[b]).wait()

    def s_descr(j, b):
        return bufs.at[b], out_hbm.at[pl.ds(row_base + j * _CHUNK, _CHUNK)]

    def s_start(j, b):
        src, dst = s_descr(j, b)
        pltpu.async_copy(src, dst, ssems.at[b])

    def s_wait(j, b):
        src, dst = s_descr(j, b)
        pltpu.make_async_copy(src, dst, ssems.at[b]).wait()

    for b in range(min(_NBUF, _CHUNKS_PER_W)):
        g_start(b, b)

    for j in range(_CHUNKS_PER_W):
        b = j % _NBUF
        g_wait(j, b)
        s_start(j, b)
        jn = j + _NBUF
        if jn < _CHUNKS_PER_W:
            s_wait(j, b)
            g_start(jn, b)

    for j in range(max(0, _CHUNKS_PER_W - _NBUF), _CHUNKS_PER_W):
        s_wait(j, j % _NBUF)


def kernel(token_ids, weight):
    idx = token_ids.T.reshape(_NW, _IDX_PER_W).astype(jnp.int32)
    flat = _gather_kernel(idx, weight)
    return flat.reshape(N_FIELDS, BATCH, EMBEDDING_DIM).transpose(1, 0, 2)
